# baseline (speedup 1.0000x reference)
import functools

import jax
import jax.numpy as jnp
from jax import lax
from jax.experimental import pallas as pl
from jax.experimental.pallas import tpu as pltpu
from jax.experimental.pallas import tpu_sc as plsc

_NUM_OBJS = 1000
_NUM_RELS = 64
_BATCH = 16384

_NC, _NS, _L = 2, 16, 16
_NW = _NC * _NS
_B_PER_W = _BATCH // _NW
_CHUNK = 128
_N_CHUNKS = _B_PER_W // _CHUNK
_IPC = _CHUNK // _L


@functools.partial(
    pl.kernel,
    out_type=jax.ShapeDtypeStruct((_BATCH, _NUM_RELS), jnp.float32),
    mesh=plsc.VectorSubcoreMesh(core_axis_name="c", subcore_axis_name="s"),
    scratch_types=[
        pltpu.VMEM((_B_PER_W,), jnp.int32),
        pltpu.VMEM((_B_PER_W,), jnp.int32),
        pltpu.VMEM((_N_CHUNKS, _CHUNK), jnp.int32),
        pltpu.VMEM((_B_PER_W, _NUM_RELS), jnp.float32),
        pltpu.SemaphoreType.DMA,
    ],
    compiler_params=pltpu.CompilerParams(use_tc_tiling_on_sc=False),
)
def _freq_bias_sc(lab0_hbm, lab1_hbm, table_hbm, out_hbm, lab0_v, lab1_v, idx_v, rows_v, sem):
    wid = lax.axis_index("s") * _NC + lax.axis_index("c")
    base = wid * _B_PER_W

    pltpu.sync_copy(lab0_hbm.at[pl.ds(base, _B_PER_W)], lab0_v)
    pltpu.sync_copy(lab1_hbm.at[pl.ds(base, _B_PER_W)], lab1_v)

    for i in range(_B_PER_W // _L):
        l0 = lab0_v[pl.ds(i * _L, _L)]
        l1 = lab1_v[pl.ds(i * _L, _L)]
        idx_v[i // _IPC, pl.ds((i % _IPC) * _L, _L)] = l0 * _NUM_OBJS + l1

    copies = []
    for j in range(_N_CHUNKS):
        copies.append(
            pltpu.async_copy(
                table_hbm.at[idx_v.at[j]],
                rows_v.at[pl.ds(j * _CHUNK, _CHUNK), :],
                sem,
            )
        )
    for c in copies:
        c.wait()

    pltpu.sync_copy(rows_v, out_hbm.at[pl.ds(base, _B_PER_W), :])


def kernel(labels, obj_baseline):
    return _freq_bias_sc(labels[:, 0], labels[:, 1], obj_baseline)
